# baseline (speedup 1.0000x reference)
import functools

import numpy as np

import jax
import jax.numpy as jnp
from jax import lax
from jax.experimental import pallas as pl
from jax.experimental.pallas import tpu as pltpu
from jax.experimental.pallas import tpu_sc as plsc

NC = 2
NS = 16
NW = NC * NS
LANES = 16
NBUF = 4


def _build_emb_kernel(B, L, D, V):
    RPW = B * L // NW
    SENT_PW = B // NW
    G1 = 128
    G2 = L - G1
    NGRP = (L + LANES - 1) // LANES

    mesh = plsc.VectorSubcoreMesh(core_axis_name="c", subcore_axis_name="s")

    @functools.partial(
        pl.kernel,
        out_type=jax.ShapeDtypeStruct((B, L, D), jnp.float32),
        mesh=mesh,
        compiler_params=pltpu.CompilerParams(
            use_tc_tiling_on_sc=False, needs_layout_passes=False),
        scratch_types=[
            pltpu.VMEM((RPW + 2 * L,), jnp.int32),
            pltpu.VMEM((SENT_PW,), jnp.int32),
            pltpu.VMEM((LANES + NGRP * LANES,), jnp.float32),
            pltpu.VMEM((NBUF, L, D), jnp.float32),
        ] + [pltpu.SemaphoreType.DMA] * (2 * NBUF),
    )
    def body(sent_ref, len_ref, table_ref, out_ref, idx_v, lens_v, mask_v,
             rows_v, *sems):
        gsem = sems[:NBUF]
        osem = sems[NBUF:]
        wid = lax.axis_index("s") * NC + lax.axis_index("c")
        sent0 = wid * SENT_PW
        pltpu.sync_copy(sent_ref.at[pl.ds(wid * RPW, RPW)],
                        idx_v.at[pl.ds(0, RPW)])
        pltpu.sync_copy(len_ref.at[pl.ds(sent0, SENT_PW)], lens_v)
        zi = jnp.full((LANES,), 0, jnp.int32)
        for g in range(2 * L // LANES):
            idx_v[pl.ds(RPW + g * LANES, LANES)] = zi

        def start_gather(j, b):
            pltpu.async_copy(table_ref.at[idx_v.at[pl.ds(j * L, G1)]],
                             rows_v.at[b, pl.ds(0, G1)], gsem[b])
            pltpu.async_copy(table_ref.at[idx_v.at[pl.ds(j * L + G1, G2)]],
                             rows_v.at[b, pl.ds(G1, G2)], gsem[b])

        def wait_gather(b):
            pltpu.make_async_copy(table_ref.at[idx_v.at[pl.ds(0, G1)]],
                                  rows_v.at[b, pl.ds(0, G1)], gsem[b]).wait()
            pltpu.make_async_copy(table_ref.at[idx_v.at[pl.ds(0, G2)]],
                                  rows_v.at[b, pl.ds(G1, G2)], gsem[b]).wait()

        def start_out(j, b):
            pltpu.async_copy(rows_v.at[b], out_ref.at[sent0 + j], osem[b])

        def wait_out(b):
            pltpu.make_async_copy(rows_v.at[b], out_ref.at[sent0],
                                  osem[b]).wait()

        def mask_multiply(j, b):
            lenv = plsc.load_gather(lens_v, [jnp.full((LANES,), j, jnp.int32)])
            iota = lax.iota(jnp.int32, LANES)
            for g in range(NGRP):
                pos = jnp.full((LANES,), g * LANES, jnp.int32) + iota
                m = (pos < lenv).astype(jnp.float32)
                mask_v[pl.ds(LANES + g * LANES, LANES)] = m
            for r in range(L):
                mv = plsc.load_gather(
                    mask_v, [jnp.full((LANES,), LANES + r, jnp.int32)])
                for h in range(D // LANES):
                    sl = pl.ds(h * LANES, LANES)
                    rows_v[b, r, sl] = rows_v[b, r, sl] * mv

        start_gather(0, 0)
        start_gather(1, 1)

        def quad_body(j4, carry):
            for bi in range(NBUF):
                j = j4 * NBUF + bi
                wait_gather(bi)
                mask_multiply(j, bi)
                nb = (bi + 2) % NBUF
                if bi >= 2:
                    wait_out(nb)
                else:
                    @pl.when(j4 > 0)
                    def _():
                        wait_out(nb)

                start_gather(j + 2, nb)
                start_out(j, bi)
            return carry

        lax.fori_loop(0, SENT_PW // NBUF, quad_body, 0)
        wait_gather(0)
        wait_gather(1)
        wait_out(2)
        wait_out(3)

    return body


def kernel(sentences, sent_lengths, table):
    B, L = sentences.shape
    V, D = table.shape
    return _build_emb_kernel(B, L, D, V)(
        sentences.reshape(-1), sent_lengths, table)


# speedup vs baseline: 1.0247x; 1.0247x over previous
import functools

import jax
import jax.numpy as jnp
from jax import lax
from jax.experimental import pallas as pl
from jax.experimental.pallas import tpu as pltpu
from jax.experimental.pallas import tpu_sc as plsc

NC = 2
NS = 16
NW = NC * NS
LANES = 16
NBUF = 4
TILE_D = 8
TILE_B = 128


def _build_emb_kernel(B, L, D, V):
    BPW = B // NW
    DG = D // TILE_D
    NTILE = DG * (B // TILE_B)
    NBG = TILE_B // LANES

    mesh = plsc.VectorSubcoreMesh(core_axis_name="c", subcore_axis_name="s")

    @functools.partial(
        pl.kernel,
        out_type=jax.ShapeDtypeStruct((L, NTILE, TILE_D * TILE_B), jnp.float32),
        mesh=mesh,
        compiler_params=pltpu.CompilerParams(
            use_tc_tiling_on_sc=False, needs_layout_passes=False),
        scratch_types=[
            pltpu.VMEM((L + 2, TILE_B), jnp.int32),
            pltpu.VMEM((BPW,), jnp.int32),
            pltpu.VMEM(((NBUF + 1) * TILE_B, D), jnp.float32),
            pltpu.VMEM((NBUF, D * TILE_B), jnp.float32),
        ] + [pltpu.SemaphoreType.DMA] * (2 * NBUF),
    )
    def body(sent_ref, len_ref, table_ref, out_ref, idx_v, lens_v, rows_v,
             ow_v, *sems):
        gsem = sems[:NBUF]
        osem = sems[NBUF:]
        wid = lax.axis_index("s") * NC + lax.axis_index("c")
        pltpu.sync_copy(sent_ref.at[:, pl.ds(wid * BPW, BPW)],
                        idx_v.at[pl.ds(0, L)])
        pltpu.sync_copy(len_ref.at[pl.ds(wid * BPW, BPW)], lens_v)
        zi = jnp.full((LANES,), 0, jnp.int32)
        for k in range(2):
            for g in range(TILE_B // LANES):
                idx_v[L + k, pl.ds(g * LANES, LANES)] = zi

        def start_gather(l, b):
            pltpu.async_copy(table_ref.at[idx_v.at[l]],
                             rows_v.at[pl.ds((b + 1) * TILE_B, TILE_B)],
                             gsem[b])

        def wait_gather(b):
            pltpu.make_async_copy(table_ref.at[idx_v.at[0]],
                                  rows_v.at[pl.ds((b + 1) * TILE_B, TILE_B)],
                                  gsem[b]).wait()

        def start_out(l, b):
            for dg in range(DG):
                pltpu.async_copy(
                    ow_v.at[b, pl.ds(dg * TILE_D * TILE_B, TILE_D * TILE_B)],
                    out_ref.at[l, dg * (B // TILE_B) + wid], osem[b])

        def wait_out(b):
            for dg in range(DG):
                pltpu.make_async_copy(
                    ow_v.at[b, pl.ds(dg * TILE_D * TILE_B, TILE_D * TILE_B)],
                    out_ref.at[0, dg * (B // TILE_B) + wid], osem[b]).wait()

        iota = lax.iota(jnp.int32, LANES)

        def transpose_mask(l, b):
            lv = jnp.full((LANES,), l, jnp.int32)
            masks = []
            for bg in range(NBG):
                lens16 = lens_v[pl.ds(bg * LANES, LANES)]
                masks.append((lv < lens16).astype(jnp.float32))
            row0 = (b + 1) * TILE_B
            for dg in range(DG):
                for ds_ in range(TILE_D):
                    f = dg * TILE_D + ds_
                    for bg in range(NBG):
                        ridx = jnp.full((LANES,), row0 + bg * LANES,
                                        jnp.int32) + iota
                        cidx = jnp.full((LANES,), f, jnp.int32)
                        v = plsc.load_gather(rows_v, [ridx, cidx]) * masks[bg]
                        ow_v[b, pl.ds(dg * TILE_D * TILE_B + ds_ * TILE_B
                                      + bg * LANES, LANES)] = v

        start_gather(0, 0)
        start_gather(1, 1)

        def quad_body(l4, carry):
            for bi in range(NBUF):
                l = l4 * NBUF + bi
                wait_gather(bi)
                transpose_mask(l, bi)
                nb = (bi + 2) % NBUF
                if bi >= 2:
                    wait_out(nb)
                else:
                    @pl.when(l4 > 0)
                    def _():
                        wait_out(nb)

                start_gather(l + 2, nb)
                start_out(l, bi)
            return carry

        lax.fori_loop(0, L // NBUF, quad_body, 0)
        wait_gather(0)
        wait_gather(1)
        wait_out(2)
        wait_out(3)

    return body


def kernel(sentences, sent_lengths, table):
    B, L = sentences.shape
    V, D = table.shape
    out_pl = _build_emb_kernel(B, L, D, V)(
        sentences.T, sent_lengths, table)
    out = out_pl.reshape(L, D // TILE_D, B // TILE_B, TILE_D, TILE_B)
    out = out.transpose(2, 4, 0, 1, 3).reshape(B, L, D)
    return out


# speedup vs baseline: 1.2900x; 1.2589x over previous
import functools

import jax
import jax.numpy as jnp
from jax import lax
from jax.experimental import pallas as pl
from jax.experimental.pallas import tpu as pltpu
from jax.experimental.pallas import tpu_sc as plsc

NC = 2
NS = 16
NW = NC * NS
LANES = 16
NBUF = 4
TILE_D = 8
TILE_B = 128
PAD_B = TILE_B + 1


def _build_emb_kernel(B, L, D, V):
    BPW = B // NW
    DG = D // TILE_D
    NTILE = DG * (B // TILE_B)
    NBG = TILE_B // LANES

    mesh = plsc.VectorSubcoreMesh(core_axis_name="c", subcore_axis_name="s")

    @functools.partial(
        pl.kernel,
        out_type=jax.ShapeDtypeStruct((L, NTILE, TILE_D, TILE_B), jnp.float32),
        mesh=mesh,
        compiler_params=pltpu.CompilerParams(
            use_tc_tiling_on_sc=False, needs_layout_passes=False),
        scratch_types=[
            pltpu.VMEM((L + 2, TILE_B), jnp.int32),
            pltpu.VMEM((BPW,), jnp.int32),
            pltpu.VMEM((LANES + TILE_B,), jnp.float32),
            pltpu.VMEM((NBUF * TILE_B, D), jnp.float32),
            pltpu.VMEM((NBUF, DG, TILE_D, PAD_B), jnp.float32),
        ] + [pltpu.SemaphoreType.DMA] * (2 * NBUF),
    )
    def body(sent_ref, len_ref, table_ref, out_ref, idx_v, lens_v, mask_v,
             rows_v, ow_v, *sems):
        gsem = sems[:NBUF]
        osem = sems[NBUF:]
        wid = lax.axis_index("s") * NC + lax.axis_index("c")
        pltpu.sync_copy(sent_ref.at[:, pl.ds(wid * BPW, BPW)],
                        idx_v.at[pl.ds(0, L)])
        pltpu.sync_copy(len_ref.at[pl.ds(wid * BPW, BPW)], lens_v)
        zi = jnp.full((LANES,), 0, jnp.int32)
        for k in range(2):
            for g in range(TILE_B // LANES):
                idx_v[L + k, pl.ds(g * LANES, LANES)] = zi

        def start_gather(l, b):
            pltpu.async_copy(table_ref.at[idx_v.at[l]],
                             rows_v.at[pl.ds(b * TILE_B, TILE_B)], gsem[b])

        def wait_gather(b):
            pltpu.make_async_copy(table_ref.at[idx_v.at[0]],
                                  rows_v.at[pl.ds(b * TILE_B, TILE_B)],
                                  gsem[b]).wait()

        def start_out(l, b):
            for dg in range(DG):
                pltpu.async_copy(ow_v.at[b, dg, :, pl.ds(0, TILE_B)],
                                 out_ref.at[l, dg * (B // TILE_B) + wid],
                                 osem[b])

        def wait_out(b):
            for dg in range(DG):
                pltpu.make_async_copy(ow_v.at[b, dg, :, pl.ds(0, TILE_B)],
                                      out_ref.at[0, dg * (B // TILE_B) + wid],
                                      osem[b]).wait()

        iota = lax.iota(jnp.int32, LANES)
        fs_vec = lax.bitwise_and(iota, jnp.full((LANES,), TILE_D - 1,
                                                jnp.int32))
        dghalf = lax.shift_right_logical(
            iota, jnp.full((LANES,), 3, jnp.int32))

        def mask_transpose(l, b):
            lv = jnp.full((LANES,), l, jnp.int32)
            for bg in range(NBG):
                lens16 = lens_v[pl.ds(bg * LANES, LANES)]
                m = (lv < lens16).astype(jnp.float32)
                mask_v[pl.ds(LANES + bg * LANES, LANES)] = m
            bsplat = jnp.full((LANES,), b, jnp.int32)
            for r in range(TILE_B):
                mv = plsc.load_gather(
                    mask_v, [jnp.full((LANES,), LANES + r, jnp.int32)])
                rsplat = jnp.full((LANES,), r, jnp.int32)
                for h in range(D // LANES):
                    v = rows_v[b * TILE_B + r, pl.ds(h * LANES, LANES)] * mv
                    dgv = dghalf + jnp.full((LANES,), h * (LANES // TILE_D),
                                            jnp.int32)
                    plsc.store_scatter(ow_v, [bsplat, dgv, fs_vec, rsplat], v)

        start_gather(0, 0)
        start_gather(1, 1)

        def quad_body(l4, carry):
            for bi in range(NBUF):
                l = l4 * NBUF + bi
                wait_gather(bi)
                mask_transpose(l, bi)
                nb = (bi + 2) % NBUF
                if bi >= 2:
                    wait_out(nb)
                else:
                    @pl.when(l4 > 0)
                    def _():
                        wait_out(nb)

                start_gather(l + 2, nb)
                start_out(l, bi)
            return carry

        lax.fori_loop(0, L // NBUF, quad_body, 0)
        wait_gather(0)
        wait_gather(1)
        wait_out(2)
        wait_out(3)

    return body


def kernel(sentences, sent_lengths, table):
    B, L = sentences.shape
    V, D = table.shape
    out_pl = _build_emb_kernel(B, L, D, V)(
        sentences.T, sent_lengths, table)
    out = out_pl.reshape(L, D // TILE_D, B // TILE_B, TILE_D, TILE_B)
    out = out.transpose(2, 4, 0, 1, 3).reshape(B, L, D)
    return out
